# baseline (speedup 1.0000x reference)
import functools

import jax
import jax.numpy as jnp
from jax import lax
from jax.experimental import pallas as pl
from jax.experimental.pallas import tpu as pltpu
from jax.experimental.pallas import tpu_sc as plsc

NC = 2
NS = 16
NW = NC * NS
LANES = 16
B = 128
SUP = 4096


def _sc_spmm(in_f, out_f, cols_dim, nnz_pad):
    nvec = cols_dim // LANES
    rows_per_w = out_f // NW
    mesh = plsc.VectorSubcoreMesh(core_axis_name="c", subcore_axis_name="s")

    @functools.partial(
        pl.kernel,
        out_type=jax.ShapeDtypeStruct((NW, (out_f // NW) * cols_dim),
                                      jnp.float32),
        mesh=mesh,
        scratch_types=[
            pltpu.VMEM((rows_per_w * cols_dim,), jnp.float32),
            pltpu.VMEM((SUP,), jnp.int32),
            pltpu.VMEM((SUP,), jnp.int32),
            pltpu.VMEM((SUP,), jnp.float32),
            pltpu.VMEM((B, cols_dim), jnp.float32),
            pltpu.VMEM((B, cols_dim), jnp.float32),
            pltpu.VMEM((LANES,), jnp.int32),
            pltpu.VMEM((LANES,), jnp.int32),
            pltpu.VMEM((rows_per_w, LANES), jnp.float32),
            pltpu.SemaphoreType.DMA,
            pltpu.SemaphoreType.DMA,
            pltpu.SemaphoreType.DMA,
            pltpu.SemaphoreType.DMA,
        ],
    )
    def k(x_hbm, rows_hbm, cols_hbm, vals_hbm, bias_hbm, out_hbm,
          acc_loc, rows_a, cols_a, vals_a, xbuf0, xbuf1,
          probe_a, probe_b, bias_v, sem0, sem1, sema, semb):
        c = lax.axis_index("c")
        s = lax.axis_index("s")
        t = c * NS + s
        row0 = t * rows_per_w
        row_end = row0 + rows_per_w
        last_loc = rows_per_w - 1
        zeros16 = jnp.zeros((LANES,), jnp.float32)
        lane_iota = lax.iota(jnp.int32, LANES)

        def zrow(i, _):
            for kk in range(nvec):
                acc_loc[pl.ds(i * cols_dim + kk * LANES, LANES)] = zeros16
            return 0
        lax.fori_loop(0, rows_per_w, zrow, 0)

        def lb_probe(bounds, probe, sem):
            lo_b, hi_b = bounds
            step = (hi_b - lo_b) // 17
            idxv = jnp.minimum(lo_b + (lane_iota + 1) * step,
                               jnp.int32(nnz_pad - 1))
            return pltpu.async_copy(rows_hbm.at[idxv], probe, sem)

        def lb_update(bounds, tgt, probe):
            lo_b, hi_b = bounds
            step = (hi_b - lo_b) // 17
            pv = probe[pl.ds(0, LANES)]
            cc = jnp.int32(0)
            for j in range(LANES):
                cc = cc + jnp.where(pv[j] < tgt, 1, 0)
            lo_n = lo_b + cc * step
            hi_n = jnp.where(cc == LANES, hi_b, lo_b + (cc + 1) * step)
            return lo_n, hi_n

        def lb_final(bounds, tgt, probe, sem):
            lo_b, hi_b = bounds
            idxv = jnp.minimum(lo_b + lane_iota, jnp.int32(nnz_pad - 1))
            pltpu.async_copy(rows_hbm.at[idxv], probe, sem).wait()
            pv = probe[pl.ds(0, LANES)]
            cc = jnp.int32(0)
            for j in range(LANES):
                take = (pv[j] < tgt) & (lo_b + j < hi_b)
                cc = cc + jnp.where(take, 1, 0)
            return lo_b + cc

        ba = (jnp.int32(0), jnp.int32(nnz_pad))
        bb = (jnp.int32(0), jnp.int32(nnz_pad))
        for _ in range(5):
            da = lb_probe(ba, probe_a, sema)
            db = lb_probe(bb, probe_b, semb)
            da.wait()
            db.wait()
            ba = lb_update(ba, row0, probe_a)
            bb = lb_update(bb, row_end, probe_b)
        lo = lb_final(ba, row0, probe_a, sema)
        hi = lb_final(bb, row_end, probe_b, semb)

        start = (lo // 8) * 8
        total = hi - start
        nsup = (total + SUP - 1) // SUP

        def gather(boff, xbuf, sem):
            return pltpu.async_copy(
                x_hbm.at[cols_a.at[pl.ds(boff, B)]], xbuf, sem)

        def compute_batch(boff, xbuf, carry):
            def group(g, gcarry):
                rv = rows_a[pl.ds(boff + g * LANES, LANES)]
                vv = vals_a[pl.ds(boff + g * LANES, LANES)]
                for j in range(LANES):
                    rj = rv[j]
                    in_rng = (rj >= row0) & (rj < row_end)
                    r = jnp.where(in_rng, rj - row0, last_loc)
                    vs = jnp.where(in_rng, vv[j], 0.0)
                    off = r * cols_dim
                    vj = jnp.full((LANES,), vs, jnp.float32)
                    for kk in range(nvec):
                        plsc.addupdate(
                            acc_loc.at[pl.ds(off + kk * LANES, LANES)],
                            vj * xbuf[g * LANES + j,
                                      pl.ds(kk * LANES, LANES)])
                return gcarry

            return lax.fori_loop(0, B // LANES, group, carry)

        def super_body(sidx, carry):
            sbase = start + sidx * SUP
            rem = hi - sbase
            nb = jnp.minimum((rem + B - 1) // B, SUP // B)
            nbt = nb + (nb & 1)
            pltpu.sync_copy(rows_hbm.at[pl.ds(sbase, SUP)], rows_a)
            pltpu.sync_copy(cols_hbm.at[pl.ds(sbase, SUP)], cols_a)
            pltpu.sync_copy(vals_hbm.at[pl.ds(sbase, SUP)], vals_a)
            gather(0, xbuf0, sem0)

            def pair(i, pcarry):
                b0 = 2 * i
                pltpu.make_async_copy(
                    x_hbm.at[cols_a.at[pl.ds(b0 * B, B)]],
                    xbuf0, sem0).wait()
                gather((b0 + 1) * B, xbuf1, sem1)
                pcarry = compute_batch(b0 * B, xbuf0, pcarry)

                b1 = b0 + 1
                pltpu.make_async_copy(
                    x_hbm.at[cols_a.at[pl.ds(b1 * B, B)]],
                    xbuf1, sem1).wait()

                @pl.when(b1 + 1 < nbt)
                def _():
                    gather((b1 + 1) * B, xbuf0, sem0)
                pcarry = compute_batch(b1 * B, xbuf1, pcarry)
                return pcarry

            return lax.fori_loop(0, nbt // 2, pair, carry)

        lax.fori_loop(0, nsup, super_body, jnp.int32(0))

        pltpu.sync_copy(bias_hbm.at[pl.ds(row0, rows_per_w), :], bias_v)

        def brow(i, _):
            bvec = bias_v[i, pl.ds(0, LANES)]
            for kk in range(nvec):
                plsc.addupdate(
                    acc_loc.at[pl.ds(i * cols_dim + kk * LANES, LANES)],
                    bvec)
            return 0
        lax.fori_loop(0, rows_per_w, brow, 0)

        pltpu.sync_copy(acc_loc, out_hbm.at[t])

    return k


def kernel(x, indices, values, bias):
    in_f, cols_dim = x.shape
    out_f = bias.shape[0]
    nnz = values.shape[0]
    rows = indices[0].astype(jnp.int32)
    cols = indices[1].astype(jnp.int32)
    vals = values.astype(jnp.float32)

    nnz_pad = ((nnz + LANES - 1) // LANES) * LANES
    pad = nnz_pad + SUP + B - nnz
    rows = jnp.concatenate([rows, jnp.full((pad,), out_f - 1, jnp.int32)])
    cols = jnp.concatenate([cols, jnp.zeros((pad,), jnp.int32)])
    vals = jnp.concatenate([vals, jnp.zeros((pad,), jnp.float32)])

    bias_rep = jnp.broadcast_to(bias, (out_f, LANES))
    y = _sc_spmm(in_f, out_f, cols_dim, nnz_pad)(
        x, rows, cols, vals, bias_rep)
    return y.reshape(out_f, cols_dim)


# speedup vs baseline: 2.8257x; 2.8257x over previous
import functools

import jax
import jax.numpy as jnp
from jax import lax
from jax.experimental import pallas as pl
from jax.experimental.pallas import tpu as pltpu
from jax.experimental.pallas import tpu_sc as plsc

NC = 2
NS = 16
NW = NC * NS
LANES = 16
B = 128
SUP = 4096


def _sc_spmm(in_f, out_f, cols_dim, nnz_pad):
    nvec = cols_dim // LANES
    rows_per_w = out_f // NW
    mesh = plsc.VectorSubcoreMesh(core_axis_name="c", subcore_axis_name="s")

    @functools.partial(
        pl.kernel,
        out_type=jax.ShapeDtypeStruct((NW, (out_f // NW) * cols_dim),
                                      jnp.float32),
        mesh=mesh,
        scratch_types=[
            pltpu.VMEM((rows_per_w * cols_dim,), jnp.float32),
            pltpu.VMEM((SUP,), jnp.int32),
            pltpu.VMEM((SUP,), jnp.int32),
            pltpu.VMEM((SUP,), jnp.float32),
            pltpu.VMEM((B, cols_dim), jnp.float32),
            pltpu.VMEM((B, cols_dim), jnp.float32),
            pltpu.VMEM((LANES,), jnp.int32),
            pltpu.VMEM((LANES,), jnp.int32),
            pltpu.VMEM((rows_per_w, LANES), jnp.float32),
            pltpu.SemaphoreType.DMA,
            pltpu.SemaphoreType.DMA,
            pltpu.SemaphoreType.DMA,
            pltpu.SemaphoreType.DMA,
        ],
    )
    def k(x_hbm, rows_hbm, cols_hbm, vals_hbm, bias_hbm, out_hbm,
          acc_loc, rows_a, cols_a, vals_a, xbuf0, xbuf1,
          probe_a, probe_b, bias_v, sem0, sem1, sema, semb):
        c = lax.axis_index("c")
        s = lax.axis_index("s")
        t = c * NS + s
        row0 = t * rows_per_w
        row_end = row0 + rows_per_w
        last_loc = rows_per_w - 1
        zeros16 = jnp.zeros((LANES,), jnp.float32)
        lane_iota = lax.iota(jnp.int32, LANES)

        def zrow(i, _):
            for kk in range(nvec):
                acc_loc[pl.ds(i * cols_dim + kk * LANES, LANES)] = zeros16
            return 0
        lax.fori_loop(0, rows_per_w, zrow, 0)

        def lb_probe(bounds, probe, sem):
            lo_b, hi_b = bounds
            step = (hi_b - lo_b) // 17
            idxv = jnp.minimum(lo_b + (lane_iota + 1) * step,
                               jnp.int32(nnz_pad - 1))
            return pltpu.async_copy(rows_hbm.at[idxv], probe, sem)

        def lb_update(bounds, tgt, probe):
            lo_b, hi_b = bounds
            step = (hi_b - lo_b) // 17
            pv = probe[pl.ds(0, LANES)]
            cc = jnp.int32(0)
            for j in range(LANES):
                cc = cc + jnp.where(pv[j] < tgt, 1, 0)
            lo_n = lo_b + cc * step
            hi_n = jnp.where(cc == LANES, hi_b, lo_b + (cc + 1) * step)
            return lo_n, hi_n

        def lb_final(bounds, tgt, probe, sem):
            lo_b, hi_b = bounds
            idxv = jnp.minimum(lo_b + lane_iota, jnp.int32(nnz_pad - 1))
            pltpu.async_copy(rows_hbm.at[idxv], probe, sem).wait()
            pv = probe[pl.ds(0, LANES)]
            cc = jnp.int32(0)
            for j in range(LANES):
                take = (pv[j] < tgt) & (lo_b + j < hi_b)
                cc = cc + jnp.where(take, 1, 0)
            return lo_b + cc

        ba = (jnp.int32(0), jnp.int32(nnz_pad))
        bb = (jnp.int32(0), jnp.int32(nnz_pad))
        for _ in range(5):
            da = lb_probe(ba, probe_a, sema)
            db = lb_probe(bb, probe_b, semb)
            da.wait()
            db.wait()
            ba = lb_update(ba, row0, probe_a)
            bb = lb_update(bb, row_end, probe_b)
        lo = lb_final(ba, row0, probe_a, sema)
        hi = lb_final(bb, row_end, probe_b, semb)

        start = (lo // 8) * 8
        total = hi - start
        nsup = (total + SUP - 1) // SUP

        def gather(boff, xbuf, sem):
            return pltpu.async_copy(
                x_hbm.at[cols_a.at[pl.ds(boff, B)]], xbuf, sem)

        def compute_batch(boff, xbuf, carry):
            def group(g, gcarry):
                accs, cur_row = gcarry
                rv = rows_a[pl.ds(boff + g * LANES, LANES)]
                vv = vals_a[pl.ds(boff + g * LANES, LANES)]
                rm = rv - row0
                rv_loc = jnp.minimum(jnp.maximum(rm, 0), last_loc)
                ind = jnp.minimum(jnp.abs(rm - rv_loc), 1)
                vvm = vv * (1.0 - ind.astype(jnp.float32))
                for j in range(LANES):
                    r = rv_loc[j]
                    changed = r != cur_row
                    keep = jnp.full((LANES,),
                                    jnp.where(changed, 0.0, 1.0),
                                    jnp.float32)

                    @pl.when(changed)
                    def _(accs=accs, cur_row=cur_row):
                        off = cur_row * cols_dim
                        for kk in range(nvec):
                            plsc.addupdate(
                                acc_loc.at[pl.ds(off + kk * LANES, LANES)],
                                accs[kk])

                    vj = jnp.full((LANES,), vvm[j], jnp.float32)
                    accs = tuple(
                        accs[kk] * keep
                        + vj * xbuf[g * LANES + j, pl.ds(kk * LANES, LANES)]
                        for kk in range(nvec)
                    )
                    cur_row = r
                return accs, cur_row
                return lax.cond(fast, fast_path, slow_path, (accs, cur_row))

            return lax.fori_loop(0, B // LANES, group, carry)

        def super_body(sidx, carry):
            sbase = start + sidx * SUP
            rem = hi - sbase
            nb = jnp.minimum((rem + B - 1) // B, SUP // B)
            nbt = nb + (nb & 1)
            pltpu.sync_copy(rows_hbm.at[pl.ds(sbase, SUP)], rows_a)
            pltpu.sync_copy(cols_hbm.at[pl.ds(sbase, SUP)], cols_a)
            pltpu.sync_copy(vals_hbm.at[pl.ds(sbase, SUP)], vals_a)
            gather(0, xbuf0, sem0)

            def pair(i, pcarry):
                b0 = 2 * i
                pltpu.make_async_copy(
                    x_hbm.at[cols_a.at[pl.ds(b0 * B, B)]],
                    xbuf0, sem0).wait()
                gather((b0 + 1) * B, xbuf1, sem1)
                pcarry = compute_batch(b0 * B, xbuf0, pcarry)

                b1 = b0 + 1
                pltpu.make_async_copy(
                    x_hbm.at[cols_a.at[pl.ds(b1 * B, B)]],
                    xbuf1, sem1).wait()

                @pl.when(b1 + 1 < nbt)
                def _():
                    gather((b1 + 1) * B, xbuf0, sem0)
                pcarry = compute_batch(b1 * B, xbuf1, pcarry)
                return pcarry

            return lax.fori_loop(0, nbt // 2, pair, carry)

        acc0 = tuple(zeros16 for _ in range(nvec))
        accs, cur_row = lax.fori_loop(0, nsup, super_body,
                                      (acc0, jnp.int32(0)))

        off = jnp.clip(cur_row, 0, last_loc) * cols_dim
        for kk in range(nvec):
            plsc.addupdate(acc_loc.at[pl.ds(off + kk * LANES, LANES)],
                           accs[kk])

        pltpu.sync_copy(bias_hbm.at[pl.ds(row0, rows_per_w), :], bias_v)

        def brow(i, _):
            bvec = bias_v[i, pl.ds(0, LANES)]
            for kk in range(nvec):
                plsc.addupdate(
                    acc_loc.at[pl.ds(i * cols_dim + kk * LANES, LANES)],
                    bvec)
            return 0
        lax.fori_loop(0, rows_per_w, brow, 0)

        pltpu.sync_copy(acc_loc, out_hbm.at[t])

    return k


def kernel(x, indices, values, bias):
    in_f, cols_dim = x.shape
    out_f = bias.shape[0]
    nnz = values.shape[0]
    rows = indices[0].astype(jnp.int32)
    cols = indices[1].astype(jnp.int32)
    vals = values.astype(jnp.float32)

    nnz_pad = ((nnz + LANES - 1) // LANES) * LANES
    pad = nnz_pad + SUP + B - nnz
    rows = jnp.concatenate([rows, jnp.full((pad,), out_f - 1, jnp.int32)])
    cols = jnp.concatenate([cols, jnp.zeros((pad,), jnp.int32)])
    vals = jnp.concatenate([vals, jnp.zeros((pad,), jnp.float32)])

    bias_rep = jnp.broadcast_to(bias, (out_f, LANES))
    y = _sc_spmm(in_f, out_f, cols_dim, nnz_pad)(
        x, rows, cols, vals, bias_rep)
    return y.reshape(out_f, cols_dim)
